# Initial kernel scaffold; baseline (speedup 1.0000x reference)
import functools

import jax
import jax.numpy as jnp
from jax import lax
from jax.experimental import pallas as pl
from jax.experimental.pallas import tpu as pltpu
from jax.experimental.pallas import tpu_sc as plsc

H = 7
LANES = 16
CHUNK = 1024
GROUPS = CHUNK // LANES


@functools.lru_cache(maxsize=None)
def _make_expand(vocab: int, n: int):
    info = plsc.get_sparse_core_info()
    num_cores = info.num_cores
    nw = info.num_cores * info.num_subcores
    npw = n // nw
    assert n % nw == 0 and npw % CHUNK == 0
    cpw = npw // CHUNK

    mesh = plsc.VectorSubcoreMesh(core_axis_name="c", subcore_axis_name="s")

    @functools.partial(
        pl.kernel,
        mesh=mesh,
        out_type=jax.ShapeDtypeStruct((n * H,), jnp.float32),
        scratch_types=[
            pltpu.VMEM((vocab,), jnp.float32),
            pltpu.VMEM((CHUNK,), jnp.int32),
            pltpu.VMEM((CHUNK * H,), jnp.float32),
            pltpu.VMEM((LANES,), jnp.float32),
        ],
    )
    def expand(table_hbm, scale_hbm, ids_hbm, out_hbm, table_v, ids_v, out_v, scale_v):
        wid = lax.axis_index("s") * num_cores + lax.axis_index("c")
        base = wid * npw
        pltpu.sync_copy(scale_hbm, scale_v)
        pltpu.sync_copy(table_hbm, table_v)
        iota = lax.iota(jnp.int32, LANES)
        pats = [iota * H + j for j in range(H)]
        scales = [
            plsc.load_gather(scale_v, [jnp.full((LANES,), j, jnp.int32)])
            for j in range(H)
        ]

        def chunk_body(c, carry):
            cbase = base + c * CHUNK
            pltpu.sync_copy(ids_hbm.at[pl.ds(cbase, CHUNK)], ids_v)

            def group_body(g, gcarry):
                idv = ids_v[pl.ds(g * LANES, LANES)]
                vals = plsc.load_gather(table_v, [idv])
                obase = g * (LANES * H)
                for j in range(H):
                    plsc.store_scatter(out_v, [pats[j] + obase], vals * scales[j])
                return gcarry

            lax.fori_loop(0, GROUPS, group_body, 0)
            pltpu.sync_copy(out_v, out_hbm.at[pl.ds(cbase * H, CHUNK * H)])
            return carry

        lax.fori_loop(0, cpw, chunk_body, 0)

    return expand


def kernel(ids, frequencies, amplitudes, decay):
    B, L = ids.shape
    n = B * L
    ids_flat = ids.reshape(n).astype(jnp.int32)
    h = jnp.arange(1, H + 1, dtype=jnp.float32)
    pad = jnp.zeros((LANES - H,), jnp.float32)
    hvec = jnp.concatenate([h, pad])
    svec = jnp.concatenate([1.0 / (h ** decay), pad])
    expand = _make_expand(frequencies.shape[0], n)
    freqs = expand(frequencies, hvec, ids_flat)
    amps = expand(amplitudes, svec, ids_flat)
    return freqs.reshape(B, L, H), amps.reshape(B, L, H)


# speedup vs baseline: 14.0836x; 14.0836x over previous
import functools

import jax
import jax.numpy as jnp
from jax import lax
from jax.experimental import pallas as pl
from jax.experimental.pallas import tpu as pltpu
from jax.experimental.pallas import tpu_sc as plsc

H = 7
LANES = 16
CHUNK = 1024
GROUPS = CHUNK // LANES


@functools.lru_cache(maxsize=None)
def _make_expand(vocab: int, n: int):
    info = plsc.get_sparse_core_info()
    num_cores = info.num_cores
    nw = info.num_cores * info.num_subcores
    npw = n // nw
    assert n % nw == 0 and npw % CHUNK == 0
    cpw = npw // CHUNK

    mesh = plsc.VectorSubcoreMesh(core_axis_name="c", subcore_axis_name="s")

    @functools.partial(
        pl.kernel,
        mesh=mesh,
        out_type=jax.ShapeDtypeStruct((n * H,), jnp.float32),
        compiler_params=pltpu.CompilerParams(needs_layout_passes=False),
        scratch_types=[
            pltpu.VMEM((vocab,), jnp.float32),
            pltpu.VMEM((CHUNK,), jnp.int32),
            pltpu.VMEM((CHUNK * H,), jnp.float32),
            pltpu.VMEM((H * LANES,), jnp.float32),
        ],
    )
    def expand(table_hbm, scale_hbm, ids_hbm, out_hbm, table_v, ids_v, out_v, scale_v):
        wid = lax.axis_index("s") * num_cores + lax.axis_index("c")
        base = wid * npw
        pltpu.sync_copy(scale_hbm, scale_v)
        pltpu.sync_copy(table_hbm, table_v)
        iota = lax.iota(jnp.int32, LANES)
        pats = [iota * H + j for j in range(H)]
        scales = [scale_v[pl.ds(j * LANES, LANES)] for j in range(H)]

        def chunk_body(c, carry):
            cbase = base + c * CHUNK
            pltpu.sync_copy(ids_hbm.at[pl.ds(cbase, CHUNK)], ids_v)

            def group_body(g, gcarry):
                idv = ids_v[pl.ds(g * LANES, LANES)]
                vals = plsc.load_gather(table_v, [idv])
                obase = g * (LANES * H)
                for j in range(H):
                    plsc.store_scatter(out_v, [pats[j] + obase], vals * scales[j])
                return gcarry

            lax.fori_loop(0, GROUPS, group_body, 0)
            pltpu.sync_copy(out_v, out_hbm.at[pl.ds(cbase * H, CHUNK * H)])
            return carry

        lax.fori_loop(0, cpw, chunk_body, 0)

    return expand


def kernel(ids, frequencies, amplitudes, decay):
    B, L = ids.shape
    n = B * L
    ids_flat = ids.reshape(n).astype(jnp.int32)
    h = jnp.arange(1, H + 1, dtype=jnp.float32)
    hvec = jnp.repeat(h, LANES)
    svec = jnp.repeat(1.0 / (h ** decay), LANES)
    expand = _make_expand(frequencies.shape[0], n)
    freqs = expand(frequencies, hvec, ids_flat)
    amps = expand(amplitudes, svec, ids_flat)
    return freqs.reshape(B, L, H), amps.reshape(B, L, H)


# speedup vs baseline: 152.2642x; 10.8115x over previous
import functools

import jax
import jax.numpy as jnp
from jax import lax
from jax.experimental import pallas as pl
from jax.experimental.pallas import tpu as pltpu
from jax.experimental.pallas import tpu_sc as plsc

H = 7
LANES = 16
BB = 128


@functools.lru_cache(maxsize=None)
def _make_gather_t(vocab: int, b: int, l: int):
    info = plsc.get_sparse_core_info()
    num_cores = info.num_cores
    nw = info.num_cores * info.num_subcores
    bpw = b // nw
    assert b % nw == 0 and bpw % BB == 0
    nsb = bpw // BB
    ngroups = BB // LANES

    mesh = plsc.VectorSubcoreMesh(core_axis_name="c", subcore_axis_name="s")

    @functools.partial(
        pl.kernel,
        mesh=mesh,
        out_type=(
            jax.ShapeDtypeStruct((l, b), jnp.float32),
            jax.ShapeDtypeStruct((l, b), jnp.float32),
        ),
        compiler_params=pltpu.CompilerParams(needs_layout_passes=False),
        scratch_types=[
            pltpu.VMEM((vocab,), jnp.float32),
            pltpu.VMEM((LANES, l), jnp.int32),
            pltpu.VMEM((l, BB), jnp.float32),
        ],
    )
    def gather_t(tf_hbm, ta_hbm, ids_hbm, outf_hbm, outa_hbm, table_v, ids_v, stag_v):
        wid = lax.axis_index("s") * num_cores + lax.axis_index("c")
        iota = lax.iota(jnp.int32, LANES)

        for table_hbm, out_hbm in ((tf_hbm, outf_hbm), (ta_hbm, outa_hbm)):
            pltpu.sync_copy(table_hbm, table_v)

            def sb_body(sb, carry, out_hbm=out_hbm):
                b0 = wid * bpw + sb * BB

                def g_body(g, gcarry):
                    pltpu.sync_copy(
                        ids_hbm.at[pl.ds(b0 + g * LANES, LANES), :], ids_v
                    )

                    def l_body(ll, lcarry):
                        lv = jnp.full((LANES,), ll, jnp.int32)
                        id16 = plsc.load_gather(ids_v, [iota, lv])
                        vals = plsc.load_gather(table_v, [id16])
                        stag_v[ll, pl.ds(g * LANES, LANES)] = vals
                        return lcarry

                    lax.fori_loop(0, l, l_body, 0, unroll=4)
                    return gcarry

                lax.fori_loop(0, ngroups, g_body, 0)
                pltpu.sync_copy(stag_v, out_hbm.at[:, pl.ds(b0, BB)])
                return carry

            lax.fori_loop(0, nsb, sb_body, 0)

    return gather_t


@functools.lru_cache(maxsize=None)
def _make_expand_tc(b: int, l: int):
    BT = 512
    nb = b // BT

    def body(sc_ref, ft_ref, at_ref, of_ref, oa_ref):
        ft = ft_ref[...]
        at = at_ref[...]
        for j in range(H):
            of_ref[j] = ft * sc_ref[0, j]
            oa_ref[j] = at * sc_ref[1, j]

    return pl.pallas_call(
        body,
        grid=(nb,),
        in_specs=[
            pl.BlockSpec(memory_space=pltpu.SMEM),
            pl.BlockSpec((l, BT), lambda i: (0, i)),
            pl.BlockSpec((l, BT), lambda i: (0, i)),
        ],
        out_specs=[
            pl.BlockSpec((H, l, BT), lambda i: (0, 0, i)),
            pl.BlockSpec((H, l, BT), lambda i: (0, 0, i)),
        ],
        out_shape=[
            jax.ShapeDtypeStruct((H, l, b), jnp.float32),
            jax.ShapeDtypeStruct((H, l, b), jnp.float32),
        ],
    )


def kernel(ids, frequencies, amplitudes, decay):
    B, L = ids.shape
    ids32 = ids.astype(jnp.int32)
    h = jnp.arange(1, H + 1, dtype=jnp.float32)
    pad = jnp.zeros((1,), jnp.float32)
    scales = jnp.stack(
        [jnp.concatenate([h, pad]), jnp.concatenate([1.0 / (h ** decay), pad])]
    )
    ft, at = _make_gather_t(frequencies.shape[0], B, L)(
        frequencies, amplitudes, ids32
    )
    of, oa = _make_expand_tc(B, L)(scales, ft, at)
    return jnp.transpose(of, (2, 1, 0)), jnp.transpose(oa, (2, 1, 0))


# speedup vs baseline: 245.9450x; 1.6153x over previous
import functools

import jax
import jax.numpy as jnp
from jax import lax
from jax.experimental import pallas as pl
from jax.experimental.pallas import tpu as pltpu
from jax.experimental.pallas import tpu_sc as plsc

H = 7
LANES = 16
ROWS = 16


@functools.lru_cache(maxsize=None)
def _make_gather(vocab: int, b: int, l: int):
    info = plsc.get_sparse_core_info()
    num_cores = info.num_cores
    nw = info.num_cores * info.num_subcores
    bpw = b // nw
    assert b % nw == 0 and bpw % (2 * ROWS) == 0
    nch = bpw // ROWS
    ngrp = ROWS * l // LANES

    mesh = plsc.VectorSubcoreMesh(core_axis_name="c", subcore_axis_name="s")

    @functools.partial(
        pl.kernel,
        mesh=mesh,
        out_type=(
            jax.ShapeDtypeStruct((b, l), jnp.float32),
            jax.ShapeDtypeStruct((b, l), jnp.float32),
        ),
        compiler_params=pltpu.CompilerParams(needs_layout_passes=False),
        scratch_types=[
            pltpu.VMEM((vocab,), jnp.float32),
            pltpu.VMEM((2, ROWS, l), jnp.int32),
            pltpu.VMEM((2, ROWS, l), jnp.float32),
            pltpu.SemaphoreType.DMA,
            pltpu.SemaphoreType.DMA,
            pltpu.SemaphoreType.DMA,
            pltpu.SemaphoreType.DMA,
        ],
    )
    def gather(tf_hbm, ta_hbm, ids_hbm, outf_hbm, outa_hbm, table_v, ids_v,
               stag_v, si0, si1, so0, so1):
        wid = lax.axis_index("s") * num_cores + lax.axis_index("c")
        base = wid * bpw
        iota = lax.iota(jnp.int32, LANES)
        sin = (si0, si1)
        sout = (so0, so1)

        def ids_copy(row0, bb):
            return pltpu.make_async_copy(
                ids_hbm.at[pl.ds(row0, ROWS), :], ids_v.at[bb], sin[bb]
            )

        def compute(bb):
            def grp(g, gc):
                p = iota + g * LANES
                row = p // l
                col = p - row * l
                id16 = plsc.load_gather(ids_v.at[bb], [row, col])
                vals = plsc.load_gather(table_v, [id16])
                plsc.store_scatter(stag_v.at[bb], [row, col], vals)
                return gc

            lax.fori_loop(0, ngrp, grp, 0, unroll=4)

        for table_hbm, out_hbm in ((tf_hbm, outf_hbm), (ta_hbm, outa_hbm)):
            pltpu.sync_copy(table_hbm, table_v)

            def out_copy(row0, bb, out_hbm=out_hbm):
                return pltpu.make_async_copy(
                    stag_v.at[bb], out_hbm.at[pl.ds(row0, ROWS), :], sout[bb]
                )

            for bb in range(2):
                ids_copy(base + bb * ROWS, bb).start()

            def outer(i, carry):
                for bb in range(2):
                    c = 2 * i + bb
                    row0 = base + c * ROWS
                    ids_copy(row0, bb).wait()

                    @pl.when(i > 0)
                    def _():
                        out_copy(row0, bb).wait()

                    compute(bb)
                    out_copy(row0, bb).start()

                    @pl.when(c + 2 < nch)
                    def _():
                        ids_copy(row0 + 2 * ROWS, bb).start()

                return carry

            lax.fori_loop(0, nch // 2, outer, 0)
            for bb in range(2):
                out_copy(base + (nch - 2 + bb) * ROWS, bb).wait()

    return gather


@functools.lru_cache(maxsize=None)
def _make_expand_tc(b: int, l: int):
    BT = 512
    nb = b // BT

    def body(sc_ref, f_ref, a_ref, of_ref, oa_ref):
        ft = jnp.transpose(f_ref[...], (1, 0))
        at = jnp.transpose(a_ref[...], (1, 0))
        for j in range(H):
            of_ref[j] = ft * sc_ref[0, j]
            oa_ref[j] = at * sc_ref[1, j]

    return pl.pallas_call(
        body,
        grid=(nb,),
        in_specs=[
            pl.BlockSpec(memory_space=pltpu.SMEM),
            pl.BlockSpec((BT, l), lambda i: (i, 0)),
            pl.BlockSpec((BT, l), lambda i: (i, 0)),
        ],
        out_specs=[
            pl.BlockSpec((H, l, BT), lambda i: (0, 0, i)),
            pl.BlockSpec((H, l, BT), lambda i: (0, 0, i)),
        ],
        out_shape=[
            jax.ShapeDtypeStruct((H, l, b), jnp.float32),
            jax.ShapeDtypeStruct((H, l, b), jnp.float32),
        ],
    )


def kernel(ids, frequencies, amplitudes, decay):
    B, L = ids.shape
    ids32 = ids.astype(jnp.int32)
    h = jnp.arange(1, H + 1, dtype=jnp.float32)
    pad = jnp.zeros((1,), jnp.float32)
    scales = jnp.stack(
        [jnp.concatenate([h, pad]), jnp.concatenate([1.0 / (h ** decay), pad])]
    )
    fv, av = _make_gather(frequencies.shape[0], B, L)(
        frequencies, amplitudes, ids32
    )
    of, oa = _make_expand_tc(B, L)(scales, fv, av)
    return jnp.transpose(of, (2, 1, 0)), jnp.transpose(oa, (2, 1, 0))


# speedup vs baseline: 261.5946x; 1.0636x over previous
import functools

import jax
import jax.numpy as jnp
from jax import lax
from jax.experimental import pallas as pl
from jax.experimental.pallas import tpu as pltpu
from jax.experimental.pallas import tpu_sc as plsc

H = 7
LANES = 16
ROWS = 16


@functools.lru_cache(maxsize=None)
def _make_gather(vocab: int, b: int, l: int):
    info = plsc.get_sparse_core_info()
    num_cores = info.num_cores
    num_subcores = info.num_subcores
    assert num_cores == 2
    bpw = b // num_subcores
    assert b % num_subcores == 0 and bpw % (2 * ROWS) == 0
    nch = bpw // ROWS
    ngrp = ROWS * l // LANES

    mesh = plsc.VectorSubcoreMesh(core_axis_name="c", subcore_axis_name="s")

    @functools.partial(
        pl.kernel,
        mesh=mesh,
        out_type=(
            jax.ShapeDtypeStruct((b, l), jnp.float32),
            jax.ShapeDtypeStruct((b, l), jnp.float32),
        ),
        compiler_params=pltpu.CompilerParams(needs_layout_passes=False),
        scratch_types=[
            pltpu.VMEM((vocab,), jnp.float32),
            pltpu.VMEM((2, ROWS, l), jnp.int32),
            pltpu.VMEM((2, ROWS, l), jnp.float32),
            pltpu.SemaphoreType.DMA,
            pltpu.SemaphoreType.DMA,
            pltpu.SemaphoreType.DMA,
            pltpu.SemaphoreType.DMA,
        ],
    )
    def gather(tf_hbm, ta_hbm, ids_hbm, outf_hbm, outa_hbm, table_v, ids_v,
               stag_v, si0, si1, so0, so1):
        cid = lax.axis_index("c")
        base = lax.axis_index("s") * bpw
        iota = lax.iota(jnp.int32, LANES)
        sin = (si0, si1)
        sout = (so0, so1)

        def ids_copy(row0, bb):
            return pltpu.make_async_copy(
                ids_hbm.at[pl.ds(row0, ROWS), :], ids_v.at[bb], sin[bb]
            )

        def compute(bb):
            def grp(g, gc):
                p = iota + g * LANES
                row = p // l
                col = p - row * l
                id16 = plsc.load_gather(ids_v.at[bb], [row, col])
                vals = plsc.load_gather(table_v, [id16])
                plsc.store_scatter(stag_v.at[bb], [row, col], vals)
                return gc

            lax.fori_loop(0, ngrp, grp, 0, unroll=4)

        for t_idx, (table_hbm, out_hbm) in enumerate(
            ((tf_hbm, outf_hbm), (ta_hbm, outa_hbm))
        ):

            @pl.when(cid == t_idx)
            def _(table_hbm=table_hbm, out_hbm=out_hbm):
                pltpu.sync_copy(table_hbm, table_v)

                def out_copy(row0, bb):
                    return pltpu.make_async_copy(
                        stag_v.at[bb], out_hbm.at[pl.ds(row0, ROWS), :], sout[bb]
                    )

                for bb in range(2):
                    ids_copy(base + bb * ROWS, bb).start()

                def outer(i, carry):
                    for bb in range(2):
                        c = 2 * i + bb
                        row0 = base + c * ROWS
                        ids_copy(row0, bb).wait()

                        @pl.when(i > 0)
                        def _():
                            out_copy(row0, bb).wait()

                        compute(bb)
                        out_copy(row0, bb).start()

                        @pl.when(c + 2 < nch)
                        def _():
                            ids_copy(row0 + 2 * ROWS, bb).start()

                    return carry

                lax.fori_loop(0, nch // 2, outer, 0)
                for bb in range(2):
                    out_copy(base + (nch - 2 + bb) * ROWS, bb).wait()

    return gather


@functools.lru_cache(maxsize=None)
def _make_expand_tc(b: int, l: int):
    BT = 512
    nb = b // BT

    def body(sc_ref, f_ref, a_ref, of_ref, oa_ref):
        ft = jnp.transpose(f_ref[...], (1, 0))
        at = jnp.transpose(a_ref[...], (1, 0))
        for j in range(H):
            of_ref[j] = ft * sc_ref[0, j]
            oa_ref[j] = at * sc_ref[1, j]

    return pl.pallas_call(
        body,
        grid=(nb,),
        in_specs=[
            pl.BlockSpec(memory_space=pltpu.SMEM),
            pl.BlockSpec((BT, l), lambda i: (i, 0)),
            pl.BlockSpec((BT, l), lambda i: (i, 0)),
        ],
        out_specs=[
            pl.BlockSpec((H, l, BT), lambda i: (0, 0, i)),
            pl.BlockSpec((H, l, BT), lambda i: (0, 0, i)),
        ],
        out_shape=[
            jax.ShapeDtypeStruct((H, l, b), jnp.float32),
            jax.ShapeDtypeStruct((H, l, b), jnp.float32),
        ],
    )


def kernel(ids, frequencies, amplitudes, decay):
    B, L = ids.shape
    ids32 = ids.astype(jnp.int32)
    h = jnp.arange(1, H + 1, dtype=jnp.float32)
    pad = jnp.zeros((1,), jnp.float32)
    scales = jnp.stack(
        [jnp.concatenate([h, pad]), jnp.concatenate([1.0 / (h ** decay), pad])]
    )
    fv, av = _make_gather(frequencies.shape[0], B, L)(
        frequencies, amplitudes, ids32
    )
    of, oa = _make_expand_tc(B, L)(scales, fv, av)
    return jnp.transpose(of, (2, 1, 0)), jnp.transpose(oa, (2, 1, 0))


# speedup vs baseline: 347.1581x; 1.3271x over previous
import functools

import jax
import jax.numpy as jnp
from jax import lax
from jax.experimental import pallas as pl
from jax.experimental.pallas import tpu as pltpu
from jax.experimental.pallas import tpu_sc as plsc

H = 7
LANES = 16
ROWS = 16


@functools.lru_cache(maxsize=None)
def _make_gather(vocab: int, b: int, l: int):
    info = plsc.get_sparse_core_info()
    num_cores = info.num_cores
    num_subcores = info.num_subcores
    assert num_cores == 2
    bpw = b // num_subcores
    assert b % num_subcores == 0 and bpw % (2 * ROWS) == 0
    nch = bpw // ROWS
    ngrp = ROWS * l // LANES

    mesh = plsc.VectorSubcoreMesh(core_axis_name="c", subcore_axis_name="s")

    @functools.partial(
        pl.kernel,
        mesh=mesh,
        out_type=(
            jax.ShapeDtypeStruct((b, l), jnp.float32),
            jax.ShapeDtypeStruct((b, l), jnp.float32),
        ),
        compiler_params=pltpu.CompilerParams(needs_layout_passes=False),
        scratch_types=[
            pltpu.VMEM((vocab,), jnp.float32),
            pltpu.VMEM((ROWS, l), jnp.int32),
            pltpu.VMEM((ROWS, l), jnp.int32),
            pltpu.VMEM((ROWS, l), jnp.float32),
            pltpu.VMEM((ROWS, l), jnp.float32),
            pltpu.SemaphoreType.DMA,
            pltpu.SemaphoreType.DMA,
            pltpu.SemaphoreType.DMA,
            pltpu.SemaphoreType.DMA,
        ],
    )
    def gather(tf_hbm, ta_hbm, ids_hbm, outf_hbm, outa_hbm, table_v, ids_v0,
               ids_v1, stag_v0, stag_v1, si0, si1, so0, so1):
        cid = lax.axis_index("c")
        base = lax.axis_index("s") * bpw
        ids_bufs = (ids_v0, ids_v1)
        stag_bufs = (stag_v0, stag_v1)
        sin = (si0, si1)
        sout = (so0, so1)

        def ids_copy(row0, bb):
            return pltpu.make_async_copy(
                ids_hbm.at[pl.ds(row0, ROWS), :], ids_bufs[bb], sin[bb]
            )

        offs = [o * LANES for o in range(l // LANES)]
        if l % LANES:
            offs.append(l - LANES)

        def compute(bb):
            idsb = ids_bufs[bb]
            stgb = stag_bufs[bb]

            def rowf(r, rc):
                for off in offs:
                    id16 = idsb[r, pl.ds(off, LANES)]
                    vals = plsc.load_gather(table_v, [id16])
                    stgb[r, pl.ds(off, LANES)] = vals
                return rc

            lax.fori_loop(0, ROWS, rowf, 0, unroll=2)

        for t_idx, (table_hbm, out_hbm) in enumerate(
            ((tf_hbm, outf_hbm), (ta_hbm, outa_hbm))
        ):

            @pl.when(cid == t_idx)
            def _(table_hbm=table_hbm, out_hbm=out_hbm):
                pltpu.sync_copy(table_hbm, table_v)

                def out_copy(row0, bb):
                    return pltpu.make_async_copy(
                        stag_bufs[bb], out_hbm.at[pl.ds(row0, ROWS), :], sout[bb]
                    )

                for bb in range(2):
                    ids_copy(base + bb * ROWS, bb).start()

                def outer(i, carry):
                    for bb in range(2):
                        c = 2 * i + bb
                        row0 = base + c * ROWS
                        ids_copy(row0, bb).wait()

                        @pl.when(i > 0)
                        def _():
                            out_copy(row0, bb).wait()

                        compute(bb)
                        out_copy(row0, bb).start()

                        @pl.when(c + 2 < nch)
                        def _():
                            ids_copy(row0 + 2 * ROWS, bb).start()

                    return carry

                lax.fori_loop(0, nch // 2, outer, 0)
                for bb in range(2):
                    out_copy(base + (nch - 2 + bb) * ROWS, bb).wait()

    return gather


@functools.lru_cache(maxsize=None)
def _make_expand_tc(b: int, l: int):
    BT = 512
    nb = b // BT

    def body(sc_ref, f_ref, a_ref, of_ref, oa_ref):
        ft = jnp.transpose(f_ref[...], (1, 0))
        at = jnp.transpose(a_ref[...], (1, 0))
        for j in range(H):
            of_ref[j] = ft * sc_ref[0, j]
            oa_ref[j] = at * sc_ref[1, j]

    return pl.pallas_call(
        body,
        grid=(nb,),
        in_specs=[
            pl.BlockSpec(memory_space=pltpu.SMEM),
            pl.BlockSpec((BT, l), lambda i: (i, 0)),
            pl.BlockSpec((BT, l), lambda i: (i, 0)),
        ],
        out_specs=[
            pl.BlockSpec((H, l, BT), lambda i: (0, 0, i)),
            pl.BlockSpec((H, l, BT), lambda i: (0, 0, i)),
        ],
        out_shape=[
            jax.ShapeDtypeStruct((H, l, b), jnp.float32),
            jax.ShapeDtypeStruct((H, l, b), jnp.float32),
        ],
    )


def kernel(ids, frequencies, amplitudes, decay):
    B, L = ids.shape
    ids32 = ids.astype(jnp.int32)
    h = jnp.arange(1, H + 1, dtype=jnp.float32)
    pad = jnp.zeros((1,), jnp.float32)
    scales = jnp.stack(
        [jnp.concatenate([h, pad]), jnp.concatenate([1.0 / (h ** decay), pad])]
    )
    fv, av = _make_gather(frequencies.shape[0], B, L)(
        frequencies, amplitudes, ids32
    )
    of, oa = _make_expand_tc(B, L)(scales, fv, av)
    return jnp.transpose(of, (2, 1, 0)), jnp.transpose(oa, (2, 1, 0))


# speedup vs baseline: 354.2515x; 1.0204x over previous
import functools

import jax
import jax.numpy as jnp
from jax import lax
from jax.experimental import pallas as pl
from jax.experimental.pallas import tpu as pltpu
from jax.experimental.pallas import tpu_sc as plsc

H = 7
LANES = 16
ROWS = 16


@functools.lru_cache(maxsize=None)
def _make_gather(vocab: int, b: int, l: int):
    info = plsc.get_sparse_core_info()
    num_cores = info.num_cores
    num_subcores = info.num_subcores
    assert num_cores == 2
    bpw = b // num_subcores
    assert b % num_subcores == 0 and bpw % (2 * ROWS) == 0
    nch = bpw // ROWS
    ngrp = ROWS * l // LANES

    mesh = plsc.VectorSubcoreMesh(core_axis_name="c", subcore_axis_name="s")

    @functools.partial(
        pl.kernel,
        mesh=mesh,
        out_type=(
            jax.ShapeDtypeStruct((b, l), jnp.float32),
            jax.ShapeDtypeStruct((b, l), jnp.float32),
        ),
        compiler_params=pltpu.CompilerParams(needs_layout_passes=False),
        scratch_types=[
            pltpu.VMEM((vocab,), jnp.float32),
            pltpu.VMEM((ROWS, l), jnp.int32),
            pltpu.VMEM((ROWS, l), jnp.int32),
            pltpu.VMEM((ROWS, l), jnp.float32),
            pltpu.VMEM((ROWS, l), jnp.float32),
            pltpu.SemaphoreType.DMA,
            pltpu.SemaphoreType.DMA,
            pltpu.SemaphoreType.DMA,
            pltpu.SemaphoreType.DMA,
        ],
    )
    def gather(tf_hbm, ta_hbm, ids_hbm, outf_hbm, outa_hbm, table_v, ids_v0,
               ids_v1, stag_v0, stag_v1, si0, si1, so0, so1):
        cid = lax.axis_index("c")
        base = lax.axis_index("s") * bpw
        ids_bufs = (ids_v0, ids_v1)
        stag_bufs = (stag_v0, stag_v1)
        sin = (si0, si1)
        sout = (so0, so1)

        def ids_copy(row0, bb):
            return pltpu.make_async_copy(
                ids_hbm.at[pl.ds(row0, ROWS), :], ids_bufs[bb], sin[bb]
            )

        offs = [o * LANES for o in range(l // LANES)]
        if l % LANES:
            offs.append(l - LANES)

        def compute(bb):
            idsb = ids_bufs[bb]
            stgb = stag_bufs[bb]

            def rowf(r, rc):
                for off in offs:
                    id16 = idsb[r, pl.ds(off, LANES)]
                    vals = plsc.load_gather(table_v, [id16])
                    stgb[r, pl.ds(off, LANES)] = vals
                return rc

            lax.fori_loop(0, ROWS, rowf, 0, unroll=2)

        for t_idx, (table_hbm, out_hbm) in enumerate(
            ((tf_hbm, outf_hbm), (ta_hbm, outa_hbm))
        ):

            @pl.when(cid == t_idx)
            def _(table_hbm=table_hbm, out_hbm=out_hbm):
                pltpu.sync_copy(table_hbm, table_v)

                def out_copy(row0, bb):
                    return pltpu.make_async_copy(
                        stag_bufs[bb], out_hbm.at[pl.ds(row0, ROWS), :], sout[bb]
                    )

                for bb in range(2):
                    ids_copy(base + bb * ROWS, bb).start()

                def outer(i, carry):
                    for bb in range(2):
                        c = 2 * i + bb
                        row0 = base + c * ROWS
                        ids_copy(row0, bb).wait()

                        @pl.when(i > 0)
                        def _():
                            out_copy(row0, bb).wait()

                        compute(bb)
                        out_copy(row0, bb).start()

                        @pl.when(c + 2 < nch)
                        def _():
                            ids_copy(row0 + 2 * ROWS, bb).start()

                    return carry

                lax.fori_loop(0, nch // 2, outer, 0)
                for bb in range(2):
                    out_copy(base + (nch - 2 + bb) * ROWS, bb).wait()

    return gather


@functools.lru_cache(maxsize=None)
def _make_expand_tc(b: int, l: int):
    BT = 1024
    nb = b // BT

    def body(sc_ref, f_ref, a_ref, of_ref, oa_ref):
        ft = jnp.transpose(f_ref[...], (1, 0))
        at = jnp.transpose(a_ref[...], (1, 0))
        for j in range(H):
            of_ref[j] = ft * sc_ref[0, j]
            oa_ref[j] = at * sc_ref[1, j]

    return pl.pallas_call(
        body,
        grid=(nb,),
        in_specs=[
            pl.BlockSpec(memory_space=pltpu.SMEM),
            pl.BlockSpec((BT, l), lambda i: (i, 0)),
            pl.BlockSpec((BT, l), lambda i: (i, 0)),
        ],
        out_specs=[
            pl.BlockSpec((H, l, BT), lambda i: (0, 0, i)),
            pl.BlockSpec((H, l, BT), lambda i: (0, 0, i)),
        ],
        out_shape=[
            jax.ShapeDtypeStruct((H, l, b), jnp.float32),
            jax.ShapeDtypeStruct((H, l, b), jnp.float32),
        ],
    )


def kernel(ids, frequencies, amplitudes, decay):
    B, L = ids.shape
    ids32 = ids.astype(jnp.int32)
    h = jnp.arange(1, H + 1, dtype=jnp.float32)
    pad = jnp.zeros((1,), jnp.float32)
    scales = jnp.stack(
        [jnp.concatenate([h, pad]), jnp.concatenate([1.0 / (h ** decay), pad])]
    )
    fv, av = _make_gather(frequencies.shape[0], B, L)(
        frequencies, amplitudes, ids32
    )
    of, oa = _make_expand_tc(B, L)(scales, fv, av)
    return jnp.transpose(of, (2, 1, 0)), jnp.transpose(oa, (2, 1, 0))
